# Initial kernel scaffold; baseline (speedup 1.0000x reference)
import jax
import jax.numpy as jnp
from jax.experimental import pallas as pl


def kernel(x, weight):
    raise NotImplementedError("write your pallas kernel here")


# speedup vs baseline: 2.7731x; 2.7731x over previous
import functools

import jax
import jax.numpy as jnp
from jax.experimental import pallas as pl
from jax.experimental.pallas import tpu as pltpu

DIM = 2048
N_EXPERTS = 64
TOPK = 8
N_GROUPS = 8
GROUP_SIZE = N_EXPERTS // N_GROUPS
TOPK_GROUPS = 4
BLOCK_B = 512
NEG = -1e30


def _gate_body(x_ref, wt_ref, w_ref, i_ref):
    x = x_ref[...]
    wt = wt_ref[...]
    scores = jnp.dot(x, wt, preferred_element_type=jnp.float32)

    m = jnp.max(scores, axis=1, keepdims=True)
    e = jnp.exp(scores - m)
    p = e / jnp.sum(e, axis=1, keepdims=True)

    bb = p.shape[0]
    lane = jax.lax.broadcasted_iota(jnp.int32, (bb, N_EXPERTS), 1)
    group_of = lane // GROUP_SIZE

    work = p
    keep = jnp.zeros((bb, N_EXPERTS), jnp.bool_)
    for _ in range(TOPK_GROUPS):
        mx = jnp.max(work, axis=1, keepdims=True)
        g = jnp.min(jnp.where(work == mx, group_of, N_GROUPS),
                    axis=1, keepdims=True)
        sel = group_of == g
        keep = jnp.logical_or(keep, sel)
        work = jnp.where(sel, NEG, work)

    ms = jnp.where(keep, p, 0.0)

    idx_cols = []
    w_cols = []
    for _ in range(TOPK):
        mx = jnp.max(ms, axis=1, keepdims=True)
        idx = jnp.min(jnp.where(ms == mx, lane, N_EXPERTS),
                      axis=1, keepdims=True)
        sel = lane == idx
        wv = jnp.max(jnp.where(sel, p, NEG), axis=1, keepdims=True)
        idx_cols.append(idx)
        w_cols.append(wv)
        ms = jnp.where(sel, NEG, ms)

    wout = jnp.concatenate(w_cols, axis=1)
    iout = jnp.concatenate(idx_cols, axis=1)
    w_ref[...] = jnp.maximum(wout, 1e-7)
    i_ref[...] = iout


def kernel(x, weight):
    B = x.shape[0]
    grid = (B // BLOCK_B,)
    wt = weight.T
    weights, indices = pl.pallas_call(
        _gate_body,
        grid=grid,
        in_specs=[
            pl.BlockSpec((BLOCK_B, DIM), lambda i: (i, 0)),
            pl.BlockSpec((DIM, N_EXPERTS), lambda i: (0, 0)),
        ],
        out_specs=[
            pl.BlockSpec((BLOCK_B, TOPK), lambda i: (i, 0)),
            pl.BlockSpec((BLOCK_B, TOPK), lambda i: (i, 0)),
        ],
        out_shape=[
            jax.ShapeDtypeStruct((B, TOPK), jnp.float32),
            jax.ShapeDtypeStruct((B, TOPK), jnp.int32),
        ],
    )(x, wt)
    return weights, indices


# speedup vs baseline: 3.6103x; 1.3019x over previous
import functools

import jax
import jax.numpy as jnp
from jax.experimental import pallas as pl
from jax.experimental.pallas import tpu as pltpu

DIM = 2048
N_EXPERTS = 64
TOPK = 8
N_GROUPS = 8
GROUP_SIZE = N_EXPERTS // N_GROUPS
TOPK_GROUPS = 4
BLOCK_B = 512
NEG = -1e30


def _gate_body(x_ref, wt_ref, w_ref, i_ref):
    x = x_ref[...]
    wt = wt_ref[...]
    scores = jnp.dot(x, wt, preferred_element_type=jnp.float32)

    m = jnp.max(scores, axis=1, keepdims=True)
    e = jnp.exp(scores - m)
    p = e / jnp.sum(e, axis=1, keepdims=True)

    bb = p.shape[0]
    lane = jax.lax.broadcasted_iota(jnp.int32, (bb, N_EXPERTS), 1)
    lanef = lane.astype(jnp.float32)
    groupf = jnp.floor(lanef * (1.0 / GROUP_SIZE))

    work = p
    keep = jnp.zeros((bb, N_EXPERTS), jnp.bool_)
    for _ in range(TOPK_GROUPS):
        mx = jnp.max(work, axis=1, keepdims=True)
        g = jnp.min(jnp.where(work == mx, groupf, float(N_GROUPS)),
                    axis=1, keepdims=True)
        sel = groupf == g
        keep = jnp.logical_or(keep, sel)
        work = jnp.where(sel, NEG, work)

    ms = jnp.where(keep, p, 0.0)

    idx_cols = []
    w_cols = []
    for _ in range(TOPK):
        mx = jnp.max(ms, axis=1, keepdims=True)
        idx = jnp.min(jnp.where(ms == mx, lanef, float(N_EXPERTS)),
                      axis=1, keepdims=True)
        sel = lanef == idx
        wv = jnp.max(jnp.where(sel, p, NEG), axis=1, keepdims=True)
        idx_cols.append(idx)
        w_cols.append(wv)
        ms = jnp.where(sel, NEG, ms)

    wout = jnp.concatenate(w_cols, axis=1)
    iout = jnp.concatenate(idx_cols, axis=1).astype(jnp.int32)
    w_ref[...] = jnp.maximum(wout, 1e-7)
    i_ref[...] = iout


def kernel(x, weight):
    B = x.shape[0]
    grid = (B // BLOCK_B,)
    wt = weight.T
    weights, indices = pl.pallas_call(
        _gate_body,
        grid=grid,
        in_specs=[
            pl.BlockSpec((BLOCK_B, DIM), lambda i: (i, 0)),
            pl.BlockSpec((DIM, N_EXPERTS), lambda i: (0, 0)),
        ],
        out_specs=[
            pl.BlockSpec((BLOCK_B, TOPK), lambda i: (i, 0)),
            pl.BlockSpec((BLOCK_B, TOPK), lambda i: (i, 0)),
        ],
        out_shape=[
            jax.ShapeDtypeStruct((B, TOPK), jnp.float32),
            jax.ShapeDtypeStruct((B, TOPK), jnp.int32),
        ],
    )(x, wt)
    return weights, indices


# speedup vs baseline: 5.1325x; 1.4216x over previous
import functools

import jax
import jax.numpy as jnp
from jax import lax
from jax.experimental import pallas as pl
from jax.experimental.pallas import tpu as pltpu
from jax.experimental.pallas import tpu_sc as plsc

DIM = 2048
N_EXPERTS = 64
TOPK = 8
N_GROUPS = 8
GROUP_SIZE = N_EXPERTS // N_GROUPS
TOPK_GROUPS = 4
NW = 32
LANES = 16
TC_BLOCK = 512
NEG = -1e30


def _scores_body(x_ref, w_ref, s_ref, lse_ref):
    x = x_ref[...]
    w = w_ref[...]
    st = lax.dot_general(w, x, (((1,), (1,)), ((), ())),
                         preferred_element_type=jnp.float32)
    mx = jnp.max(st, axis=0, keepdims=True)
    ssum = jnp.sum(jnp.exp(st - mx), axis=0, keepdims=True)
    s_ref[0] = st
    lse_ref[0] = mx + jnp.log(ssum)


def _route_body(s_hbm, lse_hbm, w_hbm, i_hbm, sv, lv, wv, iv):
    wid = lax.axis_index("s") * 2 + lax.axis_index("c")
    pltpu.sync_copy(s_hbm.at[wid], sv)
    pltpu.sync_copy(lse_hbm.at[wid], lv)

    tpw = sv.shape[1]
    ti = lax.iota(jnp.int32, LANES)

    def batch(b, carry):
        t = b * LANES
        tok = t + ti
        lsev = lv[0, pl.ds(t, LANES)]

        gm, ga = [], []
        for g in range(N_GROUPS):
            m = sv[g * GROUP_SIZE, pl.ds(t, LANES)]
            a = jnp.full((LANES,), g * GROUP_SIZE, jnp.int32)
            for j in range(1, GROUP_SIZE):
                e = g * GROUP_SIZE + j
                v = sv[e, pl.ds(t, LANES)]
                bt = v > m
                m = jnp.where(bt, v, m)
                a = jnp.where(bt, e, a)
            gm.append(m)
            ga.append(a)

        gmw = list(gm)
        kept = [None] * N_GROUPS
        for _ in range(TOPK_GROUPS):
            bv = gmw[0]
            bg = jnp.zeros((LANES,), jnp.int32)
            for g in range(1, N_GROUPS):
                bt = gmw[g] > bv
                bv = jnp.where(bt, gmw[g], bv)
                bg = jnp.where(bt, g, bg)
            for g in range(N_GROUPS):
                hit = bg == g
                kept[g] = hit if kept[g] is None else jnp.logical_or(kept[g], hit)
                gmw[g] = jnp.where(hit, NEG, gmw[g])

        gmk = [jnp.where(kept[g], gm[g], NEG) for g in range(N_GROUPS)]
        gak = list(ga)

        negs = jnp.full((LANES,), NEG, jnp.float32)
        for k in range(TOPK):
            bv = gmk[0]
            bg = jnp.zeros((LANES,), jnp.int32)
            be = gak[0]
            for g in range(1, N_GROUPS):
                bt = gmk[g] > bv
                bv = jnp.where(bt, gmk[g], bv)
                bg = jnp.where(bt, g, bg)
                be = jnp.where(bt, gak[g], be)
            wgt = jnp.maximum(jnp.exp(bv - lsev), 1e-7)
            wv[k, pl.ds(t, LANES)] = wgt
            iv[k, pl.ds(t, LANES)] = be
            plsc.store_scatter(sv, [be, tok], negs)
            base = bg * GROUP_SIZE
            nm = negs
            na = jnp.zeros((LANES,), jnp.int32)
            for j in range(GROUP_SIZE):
                ridx = base + j
                v = plsc.load_gather(sv, [ridx, tok])
                bt = v > nm
                nm = jnp.where(bt, v, nm)
                na = jnp.where(bt, ridx, na)
            for g in range(N_GROUPS):
                hit = bg == g
                gmk[g] = jnp.where(hit, nm, gmk[g])
                gak[g] = jnp.where(hit, na, gak[g])
        return carry

    lax.fori_loop(0, tpw // LANES, batch, 0)

    base = wid * tpw
    pltpu.sync_copy(wv, w_hbm.at[:, pl.ds(base, tpw)])
    pltpu.sync_copy(iv, i_hbm.at[:, pl.ds(base, tpw)])


def kernel(x, weight):
    B = x.shape[0]
    tpw = B // NW
    grid = (B // TC_BLOCK,)
    blocks_per_worker = tpw // TC_BLOCK

    scores, lse = pl.pallas_call(
        _scores_body,
        grid=grid,
        in_specs=[
            pl.BlockSpec((TC_BLOCK, DIM), lambda i: (i, 0)),
            pl.BlockSpec((N_EXPERTS, DIM), lambda i: (0, 0)),
        ],
        out_specs=[
            pl.BlockSpec((1, N_EXPERTS, TC_BLOCK), lambda i: (i, 0, 0)),
            pl.BlockSpec((1, 1, TC_BLOCK), lambda i: (i, 0, 0)),
        ],
        out_shape=[
            jax.ShapeDtypeStruct((B // TC_BLOCK, N_EXPERTS, TC_BLOCK),
                                 jnp.float32),
            jax.ShapeDtypeStruct((B // TC_BLOCK, 1, TC_BLOCK), jnp.float32),
        ],
    )(x, weight)

    del blocks_per_worker
    mesh = plsc.VectorSubcoreMesh(core_axis_name="c", subcore_axis_name="s")
    route = functools.partial(
        pl.kernel,
        out_type=[
            jax.ShapeDtypeStruct((TOPK, B), jnp.float32),
            jax.ShapeDtypeStruct((TOPK, B), jnp.int32),
        ],
        mesh=mesh,
        compiler_params=pltpu.CompilerParams(needs_layout_passes=False),
        scratch_types=[
            pltpu.VMEM((N_EXPERTS, tpw), jnp.float32),
            pltpu.VMEM((1, tpw), jnp.float32),
            pltpu.VMEM((TOPK, tpw), jnp.float32),
            pltpu.VMEM((TOPK, tpw), jnp.int32),
        ],
    )(_route_body)
    weights_t, indices_t = route(scores.reshape(NW, N_EXPERTS, tpw),
                                 lse.reshape(NW, 1, tpw))
    return weights_t.T, indices_t.T
